# baseline (speedup 1.0000x reference)
import functools

import jax
import jax.numpy as jnp
from jax import lax
from jax.experimental import pallas as pl
from jax.experimental.pallas import tpu as pltpu
from jax.experimental.pallas import tpu_sc as plsc

BATCH = 2
SEQ = 2048
D_MODEL = 1024
N_HEADS = 16
HEAD_DIM = 64
N_HASHES = 2
N_BUCKETS = 64

NI = BATCH * N_HEADS
SROWS = N_HASHES * SEQ
CHUNK = 32
NCHUNK = SROWS // CHUNK
NBKT = N_HASHES * N_BUCKETS

_f32 = jnp.float32
_i32 = jnp.int32


def _rotation_matrix():
    rot = jax.random.normal(jax.random.PRNGKey(0),
                            (HEAD_DIM, N_HASHES, N_BUCKETS // 2)).astype(_f32)
    r0 = rot[:, 0, :]
    r1 = rot[:, 1, :]
    return jnp.concatenate([r0, -r0, r1, -r1], axis=1)


def _proj_body(xq, xkv, wq, bq, wv, bv, qvo):
    q = jnp.dot(xq[0], wq[0], preferred_element_type=_f32) + bq[0]
    v = jnp.dot(xkv[0], wv[0], preferred_element_type=_f32) + bv[0]
    qvo[0, 0] = jnp.concatenate([q, v], axis=1)


def _project(inputs_q, inputs_kv, W_query, b_query, W_value, b_value):
    TB = 512
    grid = (BATCH, SEQ // TB, N_HEADS)
    wq = jnp.transpose(W_query, (1, 0, 2))
    wv = jnp.transpose(W_value, (1, 0, 2))
    return pl.pallas_call(
        _proj_body,
        grid=grid,
        in_specs=[
            pl.BlockSpec((1, TB, D_MODEL), lambda b, t, h: (b, t, 0)),
            pl.BlockSpec((1, TB, D_MODEL), lambda b, t, h: (b, t, 0)),
            pl.BlockSpec((1, D_MODEL, HEAD_DIM), lambda b, t, h: (h, 0, 0)),
            pl.BlockSpec((1, 1, HEAD_DIM), lambda b, t, h: (h, 0, 0)),
            pl.BlockSpec((1, D_MODEL, HEAD_DIM), lambda b, t, h: (h, 0, 0)),
            pl.BlockSpec((1, 1, HEAD_DIM), lambda b, t, h: (h, 0, 0)),
        ],
        out_specs=pl.BlockSpec((1, 1, TB, 2 * HEAD_DIM), lambda b, t, h: (b, h, t, 0)),
        out_shape=jax.ShapeDtypeStruct((BATCH, N_HEADS, SEQ, 2 * HEAD_DIM), _f32),
    )(inputs_q, inputs_kv, wq, b_query.reshape(N_HEADS, 1, HEAD_DIM),
      wv, b_value.reshape(N_HEADS, 1, HEAD_DIM))


def _argmax_first(c, width):
    mx = jnp.max(c, axis=1, keepdims=True)
    ii = lax.broadcasted_iota(_i32, c.shape, 1)
    hit = jnp.where(c == mx, ii, width)
    return jnp.min(hit, axis=1, keepdims=True)


def _hashsort_body(qr, rr, sgo):
    inst = pl.program_id(0)
    q = qr[0][:, :HEAD_DIM]
    m = jnp.dot(q, rr[...], preferred_element_type=_f32)
    b0 = _argmax_first(m[:, :64], 64)
    b1 = _argmax_first(m[:, 64:], 64) + 64
    bkt = jnp.concatenate([b0, b1], axis=0)

    lanes = lax.broadcasted_iota(_i32, (SROWS, NBKT), 1)
    oh = (bkt == lanes).astype(_f32)

    G = 128
    NG = SROWS // G
    T = jnp.concatenate(
        [jnp.sum(oh[g * G:(g + 1) * G], axis=0, keepdims=True) for g in range(NG)],
        axis=0)
    gr = lax.broadcasted_iota(_i32, (NG, NG), 0)
    gc = lax.broadcasted_iota(_i32, (NG, NG), 1)
    ltri_g = (gc < gr).astype(_f32)
    Gcum = jnp.dot(ltri_g, T, preferred_element_type=_f32)
    hist = Gcum[NG - 1:NG] + T[NG - 1:NG]
    ur = lax.broadcasted_iota(_i32, (NBKT, NBKT), 0)
    uc = lax.broadcasted_iota(_i32, (NBKT, NBKT), 1)
    upper = (ur < uc).astype(_f32)
    off = jnp.dot(hist, upper, preferred_element_type=_f32)
    base = off + Gcum

    lr = lax.broadcasted_iota(_i32, (G, G), 0)
    lc = lax.broadcasted_iota(_i32, (G, G), 1)
    ltri = (lc < lr).astype(_f32)
    s_parts = []
    for g in range(NG):
        oh_g = oh[g * G:(g + 1) * G]
        P_g = jnp.dot(ltri, oh_g, preferred_element_type=_f32)
        val = base[g:g + 1] + P_g
        s_parts.append(jnp.sum(oh_g * val, axis=1, keepdims=True))
    s = jnp.concatenate(s_parts, axis=0)
    sgo[0] = s.astype(_i32) + inst * SROWS


def _hashsort(qv_all, r128):
    return pl.pallas_call(
        _hashsort_body,
        grid=(NI,),
        in_specs=[
            pl.BlockSpec((1, SEQ, 2 * HEAD_DIM), lambda w: (w, 0, 0)),
            pl.BlockSpec((HEAD_DIM, NBKT), lambda w: (0, 0)),
        ],
        out_specs=pl.BlockSpec((1, SROWS, 1), lambda w: (w, 0, 0)),
        out_shape=jax.ShapeDtypeStruct((NI, SROWS, 1), _i32),
    )(qv_all, r128)


def _sc_mesh():
    return plsc.VectorSubcoreMesh(core_axis_name="c", subcore_axis_name="s",
                                  num_cores=2, num_subcores=16)


_RPC = 128
_NCH = SROWS // _RPC


def _sc_scatter_body(qvf, sg2, sqv_o, idx_v, rows, sem):
    wid = lax.axis_index("s") * 2 + lax.axis_index("c")
    pltpu.sync_copy(sg2.at[pl.ds(wid * _NCH, _NCH)], idx_v)

    def body(j, carry):
        base = wid * SEQ + lax.rem(j, _i32(SEQ // _RPC)) * _RPC
        pltpu.sync_copy(qvf.at[pl.ds(base, _RPC)], rows)
        pltpu.async_copy(rows, sqv_o.at[idx_v.at[j]], sem).wait()
        return carry

    lax.fori_loop(0, _NCH, body, 0)


def _sc_scatter(qvf, sg2):
    run = pl.kernel(
        _sc_scatter_body,
        out_type=jax.ShapeDtypeStruct((NI * SROWS, 2 * HEAD_DIM), _f32),
        mesh=_sc_mesh(),
        scratch_types=[
            pltpu.VMEM((_NCH, _RPC), _i32),
            pltpu.VMEM((_RPC, 2 * HEAD_DIM), _f32),
            pltpu.SemaphoreType.DMA,
        ],
    )
    return run(qvf, sg2)


def _attn_body(qm, qh, xo):
    main = qm[0]
    ext = jnp.concatenate([qh[0], main], axis=0)
    q = main[:, :HEAD_DIM]
    k = ext[:, :HEAD_DIM]
    v = ext[:, HEAD_DIM:]
    dots = lax.dot_general(q, k, (((1,), (1,)), ((), ())),
                           preferred_element_type=_f32) * (float(HEAD_DIM) ** 0.5)
    TB = q.shape[0]
    KB = TB + CHUNK
    ql = lax.broadcasted_iota(_i32, (TB, KB), 0) // CHUNK
    kl = lax.broadcasted_iota(_i32, (TB, KB), 1) // CHUNK - 1
    allow = (kl == ql) | (kl == ql - 1)
    d = jnp.where(allow, dots, _f32(-1e30))
    mx = jnp.max(d, axis=1, keepdims=True)
    ex = jnp.exp(d - mx)
    sm = jnp.sum(ex, axis=1, keepdims=True)
    lse = mx + jnp.log(sm)
    p = ex / sm
    x = jnp.dot(p, v, preferred_element_type=_f32)
    xo[0] = jnp.concatenate(
        [x, jnp.broadcast_to(lse, (TB, HEAD_DIM))], axis=1)


def _attention(sqv):
    TB = 128
    CPB = TB // CHUNK
    grid = (NI, SROWS // TB)
    halo = lambda w, i: (w, (i * CPB + (NCHUNK - 1)) % NCHUNK, 0)
    return pl.pallas_call(
        _attn_body,
        grid=grid,
        in_specs=[
            pl.BlockSpec((1, TB, 2 * HEAD_DIM), lambda w, i: (w, i, 0)),
            pl.BlockSpec((1, CHUNK, 2 * HEAD_DIM), halo),
        ],
        out_specs=pl.BlockSpec((1, TB, 2 * HEAD_DIM), lambda w, i: (w, i, 0)),
        out_shape=jax.ShapeDtypeStruct((NI, SROWS, 2 * HEAD_DIM), _f32),
    )(sqv, sqv)


def _sc_gather_body(xlf, sg2, og_o, idx_v, rows, sem):
    wid = lax.axis_index("s") * 2 + lax.axis_index("c")
    pltpu.sync_copy(sg2.at[pl.ds(wid * _NCH, _NCH)], idx_v)

    def body(j, carry):
        pltpu.async_copy(xlf.at[idx_v.at[j]], rows, sem).wait()
        pltpu.sync_copy(rows, og_o.at[pl.ds(wid * SROWS + j * _RPC, _RPC)])
        return carry

    lax.fori_loop(0, _NCH, body, 0)


def _sc_gather(xlf, sg2):
    run = pl.kernel(
        _sc_gather_body,
        out_type=jax.ShapeDtypeStruct((NI * SROWS, 2 * HEAD_DIM), _f32),
        mesh=_sc_mesh(),
        scratch_types=[
            pltpu.VMEM((_NCH, _RPC), _i32),
            pltpu.VMEM((_RPC, 2 * HEAD_DIM), _f32),
            pltpu.SemaphoreType.DMA,
        ],
    )
    return run(xlf, sg2)


def _comb_body(o0r, o1r, wr, br, outr):
    h = pl.program_id(2)
    o0 = o0r[0]
    o1 = o1r[0]
    x0 = o0[:, :HEAD_DIM]
    l0 = o0[:, HEAD_DIM:HEAD_DIM + 1]
    x1 = o1[:, :HEAD_DIM]
    l1 = o1[:, HEAD_DIM:HEAD_DIM + 1]
    m = jnp.maximum(l0, l1)
    w0 = jnp.exp(l0 - m)
    w1 = jnp.exp(l1 - m)
    comb = (x0 * w0 + x1 * w1) / (w0 + w1)
    part = jnp.dot(comb, wr[0], preferred_element_type=_f32)

    @pl.when(h == 0)
    def _():
        outr[0] = part + br[...]

    @pl.when(h > 0)
    def _():
        outr[0] = outr[0] + part


def _combine_project(og, W_out, b_out):
    TB = 256
    NT = SEQ // TB
    grid = (BATCH, NT, N_HEADS)
    return pl.pallas_call(
        _comb_body,
        grid=grid,
        in_specs=[
            pl.BlockSpec((1, TB, 2 * HEAD_DIM), lambda b, t, h: (b * N_HEADS + h, t, 0)),
            pl.BlockSpec((1, TB, 2 * HEAD_DIM), lambda b, t, h: (b * N_HEADS + h, t + NT, 0)),
            pl.BlockSpec((1, HEAD_DIM, D_MODEL), lambda b, t, h: (h, 0, 0)),
            pl.BlockSpec((1, D_MODEL), lambda b, t, h: (0, 0)),
        ],
        out_specs=pl.BlockSpec((1, TB, D_MODEL), lambda b, t, h: (b, t, 0)),
        out_shape=jax.ShapeDtypeStruct((BATCH, SEQ, D_MODEL), _f32),
    )(og, og, W_out, b_out.reshape(1, D_MODEL))


def kernel(inputs_q, inputs_kv, W_query, b_query, W_value, b_value, W_out, b_out):
    r128 = _rotation_matrix()
    qv4 = _project(inputs_q, inputs_kv, W_query, b_query, W_value, b_value)
    qv_all = qv4.reshape(NI, SEQ, 2 * HEAD_DIM)
    sg = _hashsort(qv_all, r128)
    sg2 = sg.reshape(NI * _NCH, _RPC)
    sqvf = _sc_scatter(qv_all.reshape(NI * SEQ, 2 * HEAD_DIM), sg2)
    xl = _attention(sqvf.reshape(NI, SROWS, 2 * HEAD_DIM))
    og = _sc_gather(xl.reshape(NI * SROWS, 2 * HEAD_DIM), sg2)
    return _combine_project(og.reshape(NI, SROWS, 2 * HEAD_DIM), W_out, b_out)


# speedup vs baseline: 1.4771x; 1.4771x over previous
import functools

import jax
import jax.numpy as jnp
from jax import lax
from jax.experimental import pallas as pl
from jax.experimental.pallas import tpu as pltpu
from jax.experimental.pallas import tpu_sc as plsc

BATCH = 2
SEQ = 2048
D_MODEL = 1024
N_HEADS = 16
HEAD_DIM = 64
N_HASHES = 2
N_BUCKETS = 64

NI = BATCH * N_HEADS
SROWS = N_HASHES * SEQ
CHUNK = 32
NCHUNK = SROWS // CHUNK
NBKT = N_HASHES * N_BUCKETS

_f32 = jnp.float32
_i32 = jnp.int32


def _rotation_matrix():
    rot = jax.random.normal(jax.random.PRNGKey(0),
                            (HEAD_DIM, N_HASHES, N_BUCKETS // 2)).astype(_f32)
    r0 = rot[:, 0, :]
    r1 = rot[:, 1, :]
    return jnp.concatenate([r0, -r0, r1, -r1], axis=1)


def _proj_body(xq, xkv, wq, bq, wv, bv, qvo):
    q = jnp.dot(xq[0], wq[0], preferred_element_type=_f32) + bq[0]
    v = jnp.dot(xkv[0], wv[0], preferred_element_type=_f32) + bv[0]
    qvo[0, 0] = jnp.concatenate([q, v], axis=1)


def _project(inputs_q, inputs_kv, W_query, b_query, W_value, b_value):
    TB = 512
    grid = (BATCH, SEQ // TB, N_HEADS)
    wq = jnp.transpose(W_query, (1, 0, 2))
    wv = jnp.transpose(W_value, (1, 0, 2))
    return pl.pallas_call(
        _proj_body,
        grid=grid,
        in_specs=[
            pl.BlockSpec((1, TB, D_MODEL), lambda b, t, h: (b, t, 0)),
            pl.BlockSpec((1, TB, D_MODEL), lambda b, t, h: (b, t, 0)),
            pl.BlockSpec((1, D_MODEL, HEAD_DIM), lambda b, t, h: (h, 0, 0)),
            pl.BlockSpec((1, 1, HEAD_DIM), lambda b, t, h: (h, 0, 0)),
            pl.BlockSpec((1, D_MODEL, HEAD_DIM), lambda b, t, h: (h, 0, 0)),
            pl.BlockSpec((1, 1, HEAD_DIM), lambda b, t, h: (h, 0, 0)),
        ],
        out_specs=pl.BlockSpec((1, 1, TB, 2 * HEAD_DIM), lambda b, t, h: (b, h, t, 0)),
        out_shape=jax.ShapeDtypeStruct((BATCH, N_HEADS, SEQ, 2 * HEAD_DIM), _f32),
    )(inputs_q, inputs_kv, wq, b_query.reshape(N_HEADS, 1, HEAD_DIM),
      wv, b_value.reshape(N_HEADS, 1, HEAD_DIM))


def _argmax_first(c, width):
    mx = jnp.max(c, axis=1, keepdims=True)
    ii = lax.broadcasted_iota(_i32, c.shape, 1)
    hit = jnp.where(c == mx, ii, width)
    return jnp.min(hit, axis=1, keepdims=True)


def _hashsort_body(qr, rr, sgo):
    inst = pl.program_id(0)
    q = qr[0][:, :HEAD_DIM]
    m = jnp.dot(q, rr[...], preferred_element_type=_f32)
    b0 = _argmax_first(m[:, :64], 64)
    b1 = _argmax_first(m[:, 64:], 64) + 64
    bkt = jnp.concatenate([b0, b1], axis=0)

    lanes = lax.broadcasted_iota(_i32, (SROWS, NBKT), 1)
    oh = (bkt == lanes).astype(_f32)

    G = 128
    NG = SROWS // G
    T = jnp.concatenate(
        [jnp.sum(oh[g * G:(g + 1) * G], axis=0, keepdims=True) for g in range(NG)],
        axis=0)
    gr = lax.broadcasted_iota(_i32, (NG, NG), 0)
    gc = lax.broadcasted_iota(_i32, (NG, NG), 1)
    ltri_g = (gc < gr).astype(_f32)
    Gcum = jnp.dot(ltri_g, T, preferred_element_type=_f32)
    hist = Gcum[NG - 1:NG] + T[NG - 1:NG]
    ur = lax.broadcasted_iota(_i32, (NBKT, NBKT), 0)
    uc = lax.broadcasted_iota(_i32, (NBKT, NBKT), 1)
    upper = (ur < uc).astype(_f32)
    off = jnp.dot(hist, upper, preferred_element_type=_f32)
    base = off + Gcum

    lr = lax.broadcasted_iota(_i32, (G, G), 0)
    lc = lax.broadcasted_iota(_i32, (G, G), 1)
    ltri = (lc < lr).astype(_f32)
    s_parts = []
    for g in range(NG):
        oh_g = oh[g * G:(g + 1) * G]
        P_g = jnp.dot(ltri, oh_g, preferred_element_type=_f32)
        val = base[g:g + 1] + P_g
        s_parts.append(jnp.sum(oh_g * val, axis=1, keepdims=True))
    s = jnp.concatenate(s_parts, axis=0)
    sgo[0] = s.astype(_i32) + inst * SROWS


def _hashsort(qv_all, r128):
    return pl.pallas_call(
        _hashsort_body,
        grid=(NI,),
        in_specs=[
            pl.BlockSpec((1, SEQ, 2 * HEAD_DIM), lambda w: (w, 0, 0)),
            pl.BlockSpec((HEAD_DIM, NBKT), lambda w: (0, 0)),
        ],
        out_specs=pl.BlockSpec((1, SROWS, 1), lambda w: (w, 0, 0)),
        out_shape=jax.ShapeDtypeStruct((NI, SROWS, 1), _i32),
    )(qv_all, r128)


def _sc_mesh():
    return plsc.VectorSubcoreMesh(core_axis_name="c", subcore_axis_name="s",
                                  num_cores=2, num_subcores=16)


_RPC = 128
_NCH = SROWS // _RPC


def _sc_scatter_body(qvf, sg2, sqv_o, idx_v, rows, sem):
    wid = lax.axis_index("s") * 2 + lax.axis_index("c")
    pltpu.sync_copy(sg2.at[pl.ds(wid * _NCH, _NCH)], idx_v)

    def body(j, carry):
        base = wid * SEQ + lax.rem(j, _i32(SEQ // _RPC)) * _RPC
        pltpu.sync_copy(qvf.at[pl.ds(base, _RPC)], rows)
        pltpu.async_copy(rows, sqv_o.at[idx_v.at[j]], sem).wait()
        return carry

    lax.fori_loop(0, _NCH, body, 0)


def _sc_scatter(qvf, sg2):
    run = pl.kernel(
        _sc_scatter_body,
        out_type=jax.ShapeDtypeStruct((NI * SROWS, 2 * HEAD_DIM), _f32),
        mesh=_sc_mesh(),
        scratch_types=[
            pltpu.VMEM((_NCH, _RPC), _i32),
            pltpu.VMEM((_RPC, 2 * HEAD_DIM), _f32),
            pltpu.SemaphoreType.DMA,
        ],
    )
    return run(qvf, sg2)


def _attn_body(qm, qh, xo):
    main = qm[0]
    ext = jnp.concatenate([qh[0], main], axis=0)
    q = main[:, :HEAD_DIM]
    k = ext[:, :HEAD_DIM]
    v = ext[:, HEAD_DIM:]
    dots = lax.dot_general(q, k, (((1,), (1,)), ((), ())),
                           preferred_element_type=_f32) * (float(HEAD_DIM) ** 0.5)
    TB = q.shape[0]
    KB = TB + CHUNK
    ql = lax.broadcasted_iota(_i32, (TB, KB), 0) // CHUNK
    kl = lax.broadcasted_iota(_i32, (TB, KB), 1) // CHUNK - 1
    allow = (kl == ql) | (kl == ql - 1)
    d = jnp.where(allow, dots, _f32(-1e30))
    mx = jnp.max(d, axis=1, keepdims=True)
    ex = jnp.exp(d - mx)
    sm = jnp.sum(ex, axis=1, keepdims=True)
    lse = mx + jnp.log(sm)
    p = ex / sm
    x = jnp.dot(p, v, preferred_element_type=_f32)
    xo[0] = jnp.concatenate(
        [x, jnp.broadcast_to(lse, (TB, HEAD_DIM))], axis=1)


def _attention(sqv):
    TB = 512
    CPB = TB // CHUNK
    grid = (NI, SROWS // TB)
    halo = lambda w, i: (w, (i * CPB + (NCHUNK - 1)) % NCHUNK, 0)
    return pl.pallas_call(
        _attn_body,
        grid=grid,
        in_specs=[
            pl.BlockSpec((1, TB, 2 * HEAD_DIM), lambda w, i: (w, i, 0)),
            pl.BlockSpec((1, CHUNK, 2 * HEAD_DIM), halo),
        ],
        out_specs=pl.BlockSpec((1, TB, 2 * HEAD_DIM), lambda w, i: (w, i, 0)),
        out_shape=jax.ShapeDtypeStruct((NI, SROWS, 2 * HEAD_DIM), _f32),
    )(sqv, sqv)


def _sc_gather_body(xlf, sg2, og_o, idx_v, rows, sem):
    wid = lax.axis_index("s") * 2 + lax.axis_index("c")
    pltpu.sync_copy(sg2.at[pl.ds(wid * _NCH, _NCH)], idx_v)

    def body(j, carry):
        pltpu.async_copy(xlf.at[idx_v.at[j]], rows, sem).wait()
        pltpu.sync_copy(rows, og_o.at[pl.ds(wid * SROWS + j * _RPC, _RPC)])
        return carry

    lax.fori_loop(0, _NCH, body, 0)


def _sc_gather(xlf, sg2):
    run = pl.kernel(
        _sc_gather_body,
        out_type=jax.ShapeDtypeStruct((NI * SROWS, 2 * HEAD_DIM), _f32),
        mesh=_sc_mesh(),
        scratch_types=[
            pltpu.VMEM((_NCH, _RPC), _i32),
            pltpu.VMEM((_RPC, 2 * HEAD_DIM), _f32),
            pltpu.SemaphoreType.DMA,
        ],
    )
    return run(xlf, sg2)


def _comb_body(o0r, o1r, wr, br, outr):
    h = pl.program_id(2)
    o0 = o0r[0]
    o1 = o1r[0]
    x0 = o0[:, :HEAD_DIM]
    l0 = o0[:, HEAD_DIM:HEAD_DIM + 1]
    x1 = o1[:, :HEAD_DIM]
    l1 = o1[:, HEAD_DIM:HEAD_DIM + 1]
    m = jnp.maximum(l0, l1)
    w0 = jnp.exp(l0 - m)
    w1 = jnp.exp(l1 - m)
    comb = (x0 * w0 + x1 * w1) / (w0 + w1)
    part = jnp.dot(comb, wr[0], preferred_element_type=_f32)

    @pl.when(h == 0)
    def _():
        outr[0] = part + br[...]

    @pl.when(h > 0)
    def _():
        outr[0] = outr[0] + part


def _combine_project(og, W_out, b_out):
    TB = 256
    NT = SEQ // TB
    grid = (BATCH, NT, N_HEADS)
    return pl.pallas_call(
        _comb_body,
        grid=grid,
        in_specs=[
            pl.BlockSpec((1, TB, 2 * HEAD_DIM), lambda b, t, h: (b * N_HEADS + h, t, 0)),
            pl.BlockSpec((1, TB, 2 * HEAD_DIM), lambda b, t, h: (b * N_HEADS + h, t + NT, 0)),
            pl.BlockSpec((1, HEAD_DIM, D_MODEL), lambda b, t, h: (h, 0, 0)),
            pl.BlockSpec((1, D_MODEL), lambda b, t, h: (0, 0)),
        ],
        out_specs=pl.BlockSpec((1, TB, D_MODEL), lambda b, t, h: (b, t, 0)),
        out_shape=jax.ShapeDtypeStruct((BATCH, SEQ, D_MODEL), _f32),
    )(og, og, W_out, b_out.reshape(1, D_MODEL))


def kernel(inputs_q, inputs_kv, W_query, b_query, W_value, b_value, W_out, b_out):
    r128 = _rotation_matrix()
    qv4 = _project(inputs_q, inputs_kv, W_query, b_query, W_value, b_value)
    qv_all = qv4.reshape(NI, SEQ, 2 * HEAD_DIM)
    sg = _hashsort(qv_all, r128)
    sg2 = sg.reshape(NI * _NCH, _RPC)
    sqvf = _sc_scatter(qv_all.reshape(NI * SEQ, 2 * HEAD_DIM), sg2)
    xl = _attention(sqvf.reshape(NI, SROWS, 2 * HEAD_DIM))
    og = _sc_gather(xl.reshape(NI * SROWS, 2 * HEAD_DIM), sg2)
    return _combine_project(og.reshape(NI, SROWS, 2 * HEAD_DIM), W_out, b_out)


# speedup vs baseline: 1.7751x; 1.2017x over previous
import functools

import jax
import jax.numpy as jnp
from jax import lax
from jax.experimental import pallas as pl
from jax.experimental.pallas import tpu as pltpu
from jax.experimental.pallas import tpu_sc as plsc

BATCH = 2
SEQ = 2048
D_MODEL = 1024
N_HEADS = 16
HEAD_DIM = 64
N_HASHES = 2
N_BUCKETS = 64

NI = BATCH * N_HEADS
SROWS = N_HASHES * SEQ
CHUNK = 32
NCHUNK = SROWS // CHUNK
NBKT = N_HASHES * N_BUCKETS

_f32 = jnp.float32
_i32 = jnp.int32


def _rotation_matrix():
    rot = jax.random.normal(jax.random.PRNGKey(0),
                            (HEAD_DIM, N_HASHES, N_BUCKETS // 2)).astype(_f32)
    r0 = rot[:, 0, :]
    r1 = rot[:, 1, :]
    return jnp.concatenate([r0, -r0, r1, -r1], axis=1)


def _proj_body(xq, xkv, wq, bq, wv, bv, qvo):
    q = jnp.dot(xq[0], wq[0], preferred_element_type=_f32) + bq[0]
    v = jnp.dot(xkv[0], wv[0], preferred_element_type=_f32) + bv[0]
    qvo[0, 0] = jnp.concatenate([q, v], axis=1)


def _project(inputs_q, inputs_kv, W_query, b_query, W_value, b_value):
    TB = 512
    grid = (BATCH, SEQ // TB, N_HEADS)
    wq = jnp.transpose(W_query, (1, 0, 2))
    wv = jnp.transpose(W_value, (1, 0, 2))
    return pl.pallas_call(
        _proj_body,
        grid=grid,
        in_specs=[
            pl.BlockSpec((1, TB, D_MODEL), lambda b, t, h: (b, t, 0)),
            pl.BlockSpec((1, TB, D_MODEL), lambda b, t, h: (b, t, 0)),
            pl.BlockSpec((1, D_MODEL, HEAD_DIM), lambda b, t, h: (h, 0, 0)),
            pl.BlockSpec((1, 1, HEAD_DIM), lambda b, t, h: (h, 0, 0)),
            pl.BlockSpec((1, D_MODEL, HEAD_DIM), lambda b, t, h: (h, 0, 0)),
            pl.BlockSpec((1, 1, HEAD_DIM), lambda b, t, h: (h, 0, 0)),
        ],
        out_specs=pl.BlockSpec((1, 1, TB, 2 * HEAD_DIM), lambda b, t, h: (b, h, t, 0)),
        out_shape=jax.ShapeDtypeStruct((BATCH, N_HEADS, SEQ, 2 * HEAD_DIM), _f32),
    )(inputs_q, inputs_kv, wq, b_query.reshape(N_HEADS, 1, HEAD_DIM),
      wv, b_value.reshape(N_HEADS, 1, HEAD_DIM))


def _argmax_first(c, width):
    mx = jnp.max(c, axis=1, keepdims=True)
    ii = lax.broadcasted_iota(_i32, c.shape, 1)
    hit = jnp.where(c == mx, ii, width)
    return jnp.min(hit, axis=1, keepdims=True)


def _hashsort_body(qr, rr, sgo):
    inst = pl.program_id(0)
    q = qr[0][:, :HEAD_DIM]
    m = jnp.dot(q, rr[...], preferred_element_type=_f32)
    b0 = _argmax_first(m[:, :64], 64)
    b1 = _argmax_first(m[:, 64:], 64) + 64
    bkt = jnp.concatenate([b0, b1], axis=0)

    lanes = lax.broadcasted_iota(_i32, (SROWS, NBKT), 1)
    oh = (bkt == lanes).astype(_f32)

    G = 128
    NG = SROWS // G
    ar = lax.broadcasted_iota(_i32, (NG, SROWS), 0)
    ac = lax.broadcasted_iota(_i32, (NG, SROWS), 1) // G
    ind = (ar == ac).astype(_f32)
    T = jnp.dot(ind, oh, preferred_element_type=_f32)
    gr = lax.broadcasted_iota(_i32, (NG, NG), 0)
    gc = lax.broadcasted_iota(_i32, (NG, NG), 1)
    ltri_g = (gc < gr).astype(_f32)
    Gcum = jnp.dot(ltri_g, T, preferred_element_type=_f32)
    hist = Gcum[NG - 1:NG] + T[NG - 1:NG]
    ur = lax.broadcasted_iota(_i32, (NBKT, NBKT), 0)
    uc = lax.broadcasted_iota(_i32, (NBKT, NBKT), 1)
    upper = (ur < uc).astype(_f32)
    off = jnp.dot(hist, upper, preferred_element_type=_f32)
    base = off + Gcum

    lr = lax.broadcasted_iota(_i32, (G, G), 0)
    lc = lax.broadcasted_iota(_i32, (G, G), 1)
    ltri = (lc < lr).astype(_f32)
    s_parts = []
    for g in range(NG):
        oh_g = oh[g * G:(g + 1) * G]
        P_g = jnp.dot(ltri, oh_g, preferred_element_type=_f32)
        val = base[g:g + 1] + P_g
        s_parts.append(jnp.sum(oh_g * val, axis=1, keepdims=True))
    s = jnp.concatenate(s_parts, axis=0)
    sgo[0] = s.astype(_i32) + inst * SROWS


def _hashsort(qv_all, r128):
    return pl.pallas_call(
        _hashsort_body,
        grid=(NI,),
        in_specs=[
            pl.BlockSpec((1, SEQ, 2 * HEAD_DIM), lambda w: (w, 0, 0)),
            pl.BlockSpec((HEAD_DIM, NBKT), lambda w: (0, 0)),
        ],
        out_specs=pl.BlockSpec((1, SROWS, 1), lambda w: (w, 0, 0)),
        out_shape=jax.ShapeDtypeStruct((NI, SROWS, 1), _i32),
    )(qv_all, r128)


def _sc_mesh():
    return plsc.VectorSubcoreMesh(core_axis_name="c", subcore_axis_name="s",
                                  num_cores=2, num_subcores=16)


_RPC = 128
_NCH = SROWS // _RPC


def _sc_scatter_body(qvf, sg2, sqv_o, idx_v, rows, sem):
    wid = lax.axis_index("s") * 2 + lax.axis_index("c")
    pltpu.sync_copy(sg2.at[pl.ds(wid * _NCH, _NCH)], idx_v)

    def body(j, carry):
        base = wid * SEQ + lax.rem(j, _i32(SEQ // _RPC)) * _RPC
        pltpu.sync_copy(qvf.at[pl.ds(base, _RPC)], rows)
        pltpu.async_copy(rows, sqv_o.at[idx_v.at[j]], sem).wait()
        return carry

    lax.fori_loop(0, _NCH, body, 0)


def _sc_scatter(qvf, sg2):
    run = pl.kernel(
        _sc_scatter_body,
        out_type=jax.ShapeDtypeStruct((NI * SROWS, 2 * HEAD_DIM), _f32),
        mesh=_sc_mesh(),
        scratch_types=[
            pltpu.VMEM((_NCH, _RPC), _i32),
            pltpu.VMEM((_RPC, 2 * HEAD_DIM), _f32),
            pltpu.SemaphoreType.DMA,
        ],
    )
    return run(qvf, sg2)


def _attn_body(qm, qh, xo):
    main = qm[0]
    ext = jnp.concatenate([qh[0], main], axis=0)
    q = main[:, :HEAD_DIM]
    k = ext[:, :HEAD_DIM]
    v = ext[:, HEAD_DIM:]
    dots = lax.dot_general(q, k, (((1,), (1,)), ((), ())),
                           preferred_element_type=_f32) * (float(HEAD_DIM) ** 0.5)
    TB = q.shape[0]
    KB = TB + CHUNK
    ql = lax.broadcasted_iota(_i32, (TB, KB), 0) // CHUNK
    kl = lax.broadcasted_iota(_i32, (TB, KB), 1) // CHUNK - 1
    allow = (kl == ql) | (kl == ql - 1)
    d = jnp.where(allow, dots, _f32(-1e30))
    mx = jnp.max(d, axis=1, keepdims=True)
    ex = jnp.exp(d - mx)
    sm = jnp.sum(ex, axis=1, keepdims=True)
    lse = mx + jnp.log(sm)
    p = ex / sm
    x = jnp.dot(p, v, preferred_element_type=_f32)
    xo[0] = jnp.concatenate(
        [x, jnp.broadcast_to(lse, (TB, HEAD_DIM))], axis=1)


def _attention(sqv):
    TB = 512
    CPB = TB // CHUNK
    grid = (NI, SROWS // TB)
    halo = lambda w, i: (w, (i * CPB + (NCHUNK - 1)) % NCHUNK, 0)
    return pl.pallas_call(
        _attn_body,
        grid=grid,
        in_specs=[
            pl.BlockSpec((1, TB, 2 * HEAD_DIM), lambda w, i: (w, i, 0)),
            pl.BlockSpec((1, CHUNK, 2 * HEAD_DIM), halo),
        ],
        out_specs=pl.BlockSpec((1, TB, 2 * HEAD_DIM), lambda w, i: (w, i, 0)),
        out_shape=jax.ShapeDtypeStruct((NI, SROWS, 2 * HEAD_DIM), _f32),
    )(sqv, sqv)


def _sc_gather_body(xlf, sg2, og_o, idx_v, rows, sem):
    wid = lax.axis_index("s") * 2 + lax.axis_index("c")
    pltpu.sync_copy(sg2.at[pl.ds(wid * _NCH, _NCH)], idx_v)

    def body(j, carry):
        pltpu.async_copy(xlf.at[idx_v.at[j]], rows, sem).wait()
        pltpu.sync_copy(rows, og_o.at[pl.ds(wid * SROWS + j * _RPC, _RPC)])
        return carry

    lax.fori_loop(0, _NCH, body, 0)


def _sc_gather(xlf, sg2):
    run = pl.kernel(
        _sc_gather_body,
        out_type=jax.ShapeDtypeStruct((NI * SROWS, 2 * HEAD_DIM), _f32),
        mesh=_sc_mesh(),
        scratch_types=[
            pltpu.VMEM((_NCH, _RPC), _i32),
            pltpu.VMEM((_RPC, 2 * HEAD_DIM), _f32),
            pltpu.SemaphoreType.DMA,
        ],
    )
    return run(xlf, sg2)


def _comb_body(o0r, o1r, wr, br, outr):
    acc = jnp.broadcast_to(br[...], outr.shape[1:])
    for h in range(N_HEADS):
        o0 = o0r[0, h]
        o1 = o1r[0, h]
        x0 = o0[:, :HEAD_DIM]
        l0 = o0[:, HEAD_DIM:HEAD_DIM + 1]
        x1 = o1[:, :HEAD_DIM]
        l1 = o1[:, HEAD_DIM:HEAD_DIM + 1]
        m = jnp.maximum(l0, l1)
        w0 = jnp.exp(l0 - m)
        w1 = jnp.exp(l1 - m)
        comb = (x0 * w0 + x1 * w1) / (w0 + w1)
        acc = acc + jnp.dot(comb, wr[h], preferred_element_type=_f32)
    outr[0] = acc


def _combine_project(og, W_out, b_out):
    TB = 256
    NT = SEQ // TB
    grid = (BATCH, NT)
    og4 = og.reshape(BATCH, N_HEADS, SROWS, 2 * HEAD_DIM)
    return pl.pallas_call(
        _comb_body,
        grid=grid,
        in_specs=[
            pl.BlockSpec((1, N_HEADS, TB, 2 * HEAD_DIM), lambda b, t: (b, 0, t, 0)),
            pl.BlockSpec((1, N_HEADS, TB, 2 * HEAD_DIM), lambda b, t: (b, 0, t + NT, 0)),
            pl.BlockSpec((N_HEADS, HEAD_DIM, D_MODEL), lambda b, t: (0, 0, 0)),
            pl.BlockSpec((1, D_MODEL), lambda b, t: (0, 0)),
        ],
        out_specs=pl.BlockSpec((1, TB, D_MODEL), lambda b, t: (b, t, 0)),
        out_shape=jax.ShapeDtypeStruct((BATCH, SEQ, D_MODEL), _f32),
    )(og4, og4, W_out, b_out.reshape(1, D_MODEL))


def kernel(inputs_q, inputs_kv, W_query, b_query, W_value, b_value, W_out, b_out):
    r128 = _rotation_matrix()
    qv4 = _project(inputs_q, inputs_kv, W_query, b_query, W_value, b_value)
    qv_all = qv4.reshape(NI, SEQ, 2 * HEAD_DIM)
    sg = _hashsort(qv_all, r128)
    sg2 = sg.reshape(NI * _NCH, _RPC)
    sqvf = _sc_scatter(qv_all.reshape(NI * SEQ, 2 * HEAD_DIM), sg2)
    xl = _attention(sqvf.reshape(NI, SROWS, 2 * HEAD_DIM))
    og = _sc_gather(xl.reshape(NI * SROWS, 2 * HEAD_DIM), sg2)
    return _combine_project(og.reshape(NI, SROWS, 2 * HEAD_DIM), W_out, b_out)
